# baseline (speedup 1.0000x reference)
import functools

import jax
import jax.numpy as jnp
from jax import lax
from jax.experimental import pallas as pl
from jax.experimental.pallas import tpu as pltpu
from jax.experimental.pallas import tpu_sc as plsc

_NUM_CORES = 2
_NUM_CHUNKS = 4


def kernel(x, row_embed):
    b = x.shape[0]
    w = x.shape[-2]
    d = row_embed.shape[-1]
    half_rows = w // _NUM_CORES
    ch = half_rows // _NUM_CHUNKS

    mesh = plsc.ScalarSubcoreMesh(axis_name="c", num_cores=_NUM_CORES)

    @functools.partial(
        pl.kernel,
        mesh=mesh,
        out_type=jax.ShapeDtypeStruct((b * w, d), row_embed.dtype),
        scratch_types=(
            [pltpu.VMEM_SHARED((ch, d), row_embed.dtype) for _ in range(_NUM_CHUNKS)]
            + [pltpu.SemaphoreType.DMA for _ in range(_NUM_CHUNKS)]
            + [pltpu.SemaphoreType.DMA]
        ),
    )
    def _bcast(emb_hbm, out_hbm, *scratch):
        bufs = scratch[:_NUM_CHUNKS]
        sems = scratch[_NUM_CHUNKS : 2 * _NUM_CHUNKS]
        sem_w = scratch[2 * _NUM_CHUNKS]
        cid = lax.axis_index("c")
        base = cid * half_rows
        reads = [
            pltpu.async_copy(emb_hbm.at[pl.ds(base + i * ch, ch)], bufs[i], sems[i])
            for i in range(_NUM_CHUNKS)
        ]
        writes = []
        for i in range(_NUM_CHUNKS):
            reads[i].wait()
            for bb in range(b):
                writes.append(
                    pltpu.async_copy(
                        bufs[i],
                        out_hbm.at[pl.ds(bb * w + base + i * ch, ch)],
                        sem_w,
                    )
                )
        for c in writes:
            c.wait()

    return _bcast(row_embed).reshape(b, w, d)


# speedup vs baseline: 1.1722x; 1.1722x over previous
import functools

import jax
import jax.numpy as jnp
from jax import lax
from jax.experimental import pallas as pl
from jax.experimental.pallas import tpu as pltpu
from jax.experimental.pallas import tpu_sc as plsc

_NUM_CORES = 2
_NUM_SUBCORES = 16
_NUM_WORKERS = _NUM_CORES * _NUM_SUBCORES


def kernel(x, row_embed):
    b = x.shape[0]
    w = x.shape[-2]
    d = row_embed.shape[-1]
    rows_per = w // _NUM_WORKERS

    mesh = plsc.VectorSubcoreMesh(core_axis_name="c", subcore_axis_name="s")

    half = rows_per // 2

    @functools.partial(
        pl.kernel,
        mesh=mesh,
        out_type=jax.ShapeDtypeStruct((b * w, d), row_embed.dtype),
        scratch_types=[
            pltpu.VMEM((half, d), row_embed.dtype),
            pltpu.VMEM((half, d), row_embed.dtype),
            pltpu.SemaphoreType.DMA,
            pltpu.SemaphoreType.DMA,
            pltpu.SemaphoreType.DMA,
        ],
    )
    def _bcast(emb_hbm, out_hbm, buf0, buf1, sem_r0, sem_r1, sem_w):
        wid = lax.axis_index("s") * _NUM_CORES + lax.axis_index("c")
        base = wid * rows_per
        r0 = pltpu.async_copy(emb_hbm.at[pl.ds(base, half)], buf0, sem_r0)
        r1 = pltpu.async_copy(emb_hbm.at[pl.ds(base + half, half)], buf1, sem_r1)
        r0.wait()
        writes = [
            pltpu.async_copy(buf0, out_hbm.at[pl.ds(bb * w + base, half)], sem_w)
            for bb in range(b)
        ]
        r1.wait()
        writes += [
            pltpu.async_copy(buf1, out_hbm.at[pl.ds(bb * w + base + half, half)], sem_w)
            for bb in range(b)
        ]
        for c in writes:
            c.wait()

    return _bcast(row_embed).reshape(b, w, d)
